# baseline (speedup 1.0000x reference)
import functools

import jax
import jax.numpy as jnp
from jax import lax
from jax.experimental import pallas as pl
from jax.experimental.pallas import tpu as pltpu
from jax.experimental.pallas import tpu_sc as plsc

BT = 6
LANES = 16
PCH = 2048
NWORK = 32


def _tc_kernel(x_ref, t_ref, out_ref, acc_ref):
    b = pl.program_id(0)
    h = pl.program_id(1)

    x = x_ref[0]
    t = t_ref[0]
    C, Hb, W = x.shape
    cidx = jax.lax.broadcasted_iota(jnp.int32, (C, Hb, W), 0)
    mask = (t[None, :, :] == cidx).astype(jnp.float32)
    d = x - mask
    d2 = (d * d).reshape(C * Hb, W)
    ones = jnp.ones((1, C * Hb), jnp.float32)
    part = jax.lax.dot_general(
        ones, d2, (((1,), (0,)), ((), ())),
        preferred_element_type=jnp.float32)

    first = jnp.logical_and(b == 0, h == 0)

    @pl.when(first)
    def _init():
        acc_ref[...] = part

    @pl.when(jnp.logical_not(first))
    def _accum():
        acc_ref[...] += part

    @pl.when(jnp.logical_and(b == pl.num_programs(0) - 1,
                             h == pl.num_programs(1) - 1))
    def _done():
        out_ref[0] = jnp.sum(acc_ref[...])


def _tc_partial_sum(x, t, nb):
    B, C, H, W = x.shape
    HS = 2
    ssum = pl.pallas_call(
        _tc_kernel,
        grid=(nb, HS),
        in_specs=[
            pl.BlockSpec((1, C, H // HS, W), lambda b, h: (b, 0, h, 0)),
            pl.BlockSpec((1, H // HS, W), lambda b, h: (b, h, 0)),
        ],
        out_specs=pl.BlockSpec(memory_space=pltpu.SMEM),
        out_shape=jax.ShapeDtypeStruct((1,), jnp.float32),
        scratch_shapes=[pltpu.VMEM((1, W), jnp.float32)],
    )(x, t)
    return ssum[0]


def _make_sc_partial(b_start, n_batches, n_classes, H, W):
    RCW = 256
    col_halves = W // RCW
    bands = H // 8
    chunks_per_batch = bands * col_halves
    total_chunks = n_batches * chunks_per_batch
    chunks_per_worker = total_chunks // NWORK
    mesh = plsc.VectorSubcoreMesh(core_axis_name="c", subcore_axis_name="s")

    @functools.partial(
        pl.kernel, mesh=mesh,
        compiler_params=pltpu.CompilerParams(needs_layout_passes=False),
        out_type=jax.ShapeDtypeStruct((NWORK, LANES), jnp.float32),
        scratch_types=[
            pltpu.VMEM((n_classes, 8, RCW), jnp.float32),
            pltpu.VMEM((n_classes, 8, RCW), jnp.float32),
            pltpu.VMEM((8, RCW), jnp.int32),
            pltpu.VMEM((8, RCW), jnp.int32),
            pltpu.VMEM((LANES,), jnp.float32),
            pltpu.SemaphoreType.DMA,
            pltpu.SemaphoreType.DMA,
        ],
    )
    def sc_kernel(x_hbm, t_hbm, out_hbm, tile0, tile1, tv0, tv1, res_v, sem0, sem1):
        wid = lax.axis_index("s") * 2 + lax.axis_index("c")

        tiles = (tile0, tile1)
        tvs = (tv0, tv1)
        sems = (sem0, sem1)

        def chunk_coords(k):
            ci = wid * chunks_per_worker + k
            b = b_start + ci // chunks_per_batch
            r = ci % chunks_per_batch
            h0 = (r // col_halves) * 8
            w0 = (r % col_halves) * RCW
            return b, h0, w0

        def start(k, slot):
            b, h0, w0 = chunk_coords(k)
            pltpu.async_copy(
                x_hbm.at[b, :, pl.ds(h0, 8), pl.ds(w0, RCW)],
                tiles[slot], sems[slot])
            pltpu.async_copy(
                t_hbm.at[b, pl.ds(h0, 8), pl.ds(w0, RCW)],
                tvs[slot], sems[slot])

        def wait(slot):
            pltpu.make_async_copy(
                x_hbm.at[0, :, pl.ds(0, 8), pl.ds(0, RCW)],
                tiles[slot], sems[slot]).wait()
            pltpu.make_async_copy(
                t_hbm.at[0, pl.ds(0, 8), pl.ds(0, RCW)],
                tvs[slot], sems[slot]).wait()

        def consume(slot, acc):
            tile = tiles[slot]
            tv = tvs[slot]
            liota = lax.iota(jnp.int32, LANES)
            groups = RCW // LANES

            def body(r, carry):
                a1, a2 = carry
                rvec = jnp.full((LANES,), r, jnp.int32)
                for u in range(groups):
                    col = u * LANES
                    for c in range(n_classes):
                        v = tile[c, r, pl.ds(col, LANES)]
                        a1 = a1 + v * v
                    tl = tv[r, pl.ds(col, LANES)]
                    g = plsc.load_gather(tile, [tl, rvec, col + liota])
                    a2 = a2 + g
                return (a1, a2)

            return lax.fori_loop(0, 8, body, acc)

        start(0, 0)
        start(1, 1)

        def pair(k2, acc):
            k = 2 * k2
            wait(0)
            acc = consume(0, acc)
            start(k + 2, 0)
            wait(1)
            acc = consume(1, acc)
            start(k + 3, 1)
            return acc

        acc = (jnp.zeros((LANES,), jnp.float32),
               jnp.zeros((LANES,), jnp.float32))
        acc = lax.fori_loop(0, chunks_per_worker // 2 - 1, pair, acc)
        wait(0)
        acc = consume(0, acc)
        wait(1)
        acc = consume(1, acc)

        res_v[...] = acc[0] - 2.0 * acc[1]
        pltpu.sync_copy(res_v, out_hbm.at[wid])

    return sc_kernel


def kernel(output, target):
    B, C, H, W = output.shape
    target = target.astype(jnp.int32)
    n_pix = H * W

    tc_sum = _tc_partial_sum(output, target, BT)

    bsc = B - BT
    sc_parts = _make_sc_partial(BT, bsc, C, H, W)(output, target)
    sc_sum = jnp.sum(sc_parts) + jnp.float32(bsc * n_pix)

    n = B * C * H * W
    return (tc_sum + sc_sum) / jnp.float32(n)


# speedup vs baseline: 1.0939x; 1.0939x over previous
import functools

import jax
import jax.numpy as jnp
from jax import lax
from jax.experimental import pallas as pl
from jax.experimental.pallas import tpu as pltpu
from jax.experimental.pallas import tpu_sc as plsc

BT = 5
LANES = 16
PCH = 2048
NWORK = 32


def _tc_kernel(x_ref, t_ref, out_ref, acc_ref):
    b = pl.program_id(0)
    h = pl.program_id(1)

    x = x_ref[0]
    t = t_ref[0]
    C, Hb, W = x.shape
    cidx = jax.lax.broadcasted_iota(jnp.int32, (C, Hb, W), 0)
    mask = (t[None, :, :] == cidx).astype(jnp.float32)
    d = x - mask
    d2 = (d * d).reshape(C * Hb, W)
    ones = jnp.ones((1, C * Hb), jnp.float32)
    part = jax.lax.dot_general(
        ones, d2, (((1,), (0,)), ((), ())),
        preferred_element_type=jnp.float32)

    first = jnp.logical_and(b == 0, h == 0)

    @pl.when(first)
    def _init():
        acc_ref[...] = part

    @pl.when(jnp.logical_not(first))
    def _accum():
        acc_ref[...] += part

    @pl.when(jnp.logical_and(b == pl.num_programs(0) - 1,
                             h == pl.num_programs(1) - 1))
    def _done():
        out_ref[0] = jnp.sum(acc_ref[...])


def _tc_partial_sum(x, t, nb):
    B, C, H, W = x.shape
    HS = 2
    ssum = pl.pallas_call(
        _tc_kernel,
        grid=(nb, HS),
        in_specs=[
            pl.BlockSpec((1, C, H // HS, W), lambda b, h: (b, 0, h, 0)),
            pl.BlockSpec((1, H // HS, W), lambda b, h: (b, h, 0)),
        ],
        out_specs=pl.BlockSpec(memory_space=pltpu.SMEM),
        out_shape=jax.ShapeDtypeStruct((1,), jnp.float32),
        scratch_shapes=[pltpu.VMEM((1, W), jnp.float32)],
    )(x, t)
    return ssum[0]


def _make_sc_partial(b_start, n_batches, n_classes, H, W):
    RCW = 256
    col_halves = W // RCW
    bands = H // 8
    chunks_per_batch = bands * col_halves
    total_chunks = n_batches * chunks_per_batch
    chunks_per_worker = total_chunks // NWORK
    mesh = plsc.VectorSubcoreMesh(core_axis_name="c", subcore_axis_name="s")

    @functools.partial(
        pl.kernel, mesh=mesh,
        compiler_params=pltpu.CompilerParams(needs_layout_passes=False),
        out_type=jax.ShapeDtypeStruct((NWORK, LANES), jnp.float32),
        scratch_types=[
            pltpu.VMEM((n_classes, 8, RCW), jnp.float32),
            pltpu.VMEM((n_classes, 8, RCW), jnp.float32),
            pltpu.VMEM((8, RCW), jnp.int32),
            pltpu.VMEM((8, RCW), jnp.int32),
            pltpu.VMEM((LANES,), jnp.float32),
            pltpu.SemaphoreType.DMA,
            pltpu.SemaphoreType.DMA,
        ],
    )
    def sc_kernel(x_hbm, t_hbm, out_hbm, tile0, tile1, tv0, tv1, res_v, sem0, sem1):
        wid = lax.axis_index("s") * 2 + lax.axis_index("c")

        tiles = (tile0, tile1)
        tvs = (tv0, tv1)
        sems = (sem0, sem1)

        def chunk_coords(k):
            ci = wid * chunks_per_worker + k
            b = b_start + ci // chunks_per_batch
            r = ci % chunks_per_batch
            h0 = (r // col_halves) * 8
            w0 = (r % col_halves) * RCW
            return b, h0, w0

        def start(k, slot):
            b, h0, w0 = chunk_coords(k)
            pltpu.async_copy(
                x_hbm.at[b, :, pl.ds(h0, 8), pl.ds(w0, RCW)],
                tiles[slot], sems[slot])
            pltpu.async_copy(
                t_hbm.at[b, pl.ds(h0, 8), pl.ds(w0, RCW)],
                tvs[slot], sems[slot])

        def wait(slot):
            pltpu.make_async_copy(
                x_hbm.at[0, :, pl.ds(0, 8), pl.ds(0, RCW)],
                tiles[slot], sems[slot]).wait()
            pltpu.make_async_copy(
                t_hbm.at[0, pl.ds(0, 8), pl.ds(0, RCW)],
                tvs[slot], sems[slot]).wait()

        def consume(slot, acc):
            tile = tiles[slot]
            tv = tvs[slot]
            liota = lax.iota(jnp.int32, LANES)
            UNR = 2

            def body(j, carry):
                a1, a2 = carry
                r = j // (RCW // (LANES * UNR))
                base = (j % (RCW // (LANES * UNR))) * (LANES * UNR)
                rvec = jnp.full((LANES,), r, jnp.int32)
                for u in range(UNR):
                    col = base + u * LANES
                    for c in range(n_classes):
                        v = tile[c, r, pl.ds(col, LANES)]
                        a1 = a1 + v * v
                    tl = tv[r, pl.ds(col, LANES)]
                    g = plsc.load_gather(tile, [tl, rvec, col + liota])
                    a2 = a2 + g
                return (a1, a2)

            n_iters = 8 * RCW // (LANES * UNR)
            return lax.fori_loop(0, n_iters, body, acc)

        start(0, 0)
        acc = (jnp.zeros((LANES,), jnp.float32),
               jnp.zeros((LANES,), jnp.float32))
        for k in range(chunks_per_worker):
            slot = k % 2
            wait(slot)
            if k + 1 < chunks_per_worker:
                start(k + 1, (k + 1) % 2)
            acc = consume(slot, acc)

        res_v[...] = acc[0] - 2.0 * acc[1]
        pltpu.sync_copy(res_v, out_hbm.at[wid])

    return sc_kernel


def kernel(output, target):
    B, C, H, W = output.shape
    target = target.astype(jnp.int32)
    n_pix = H * W

    tc_sum = _tc_partial_sum(output, target, BT)

    bsc = B - BT
    sc_parts = _make_sc_partial(BT, bsc, C, H, W)(output, target)
    sc_sum = jnp.sum(sc_parts) + jnp.float32(bsc * n_pix)

    n = B * C * H * W
    return (tc_sum + sc_sum) / jnp.float32(n)


# speedup vs baseline: 1.5411x; 1.4088x over previous
import functools

import jax
import jax.numpy as jnp
from jax.experimental import pallas as pl
from jax.experimental.pallas import tpu as pltpu


def _mse_onehot_kernel(n_total, x_ref, t_ref, out_ref, acc_ref):
    b = pl.program_id(0)
    h = pl.program_id(1)

    x = x_ref[0]
    t = t_ref[0]
    C, Hb, W = x.shape
    cidx = jax.lax.broadcasted_iota(jnp.int32, (C, Hb, W), 0)
    mask = (t[None, :, :] == cidx).astype(jnp.float32)
    d = x - mask
    d2 = (d * d).reshape(C * Hb, W)
    ones = jnp.ones((1, C * Hb), jnp.float32)
    part = jax.lax.dot_general(
        ones, d2, (((1,), (0,)), ((), ())),
        preferred_element_type=jnp.float32)

    first = jnp.logical_and(b == 0, h == 0)

    @pl.when(first)
    def _init():
        acc_ref[...] = part

    @pl.when(jnp.logical_not(first))
    def _accum():
        acc_ref[...] += part

    @pl.when(jnp.logical_and(b == pl.num_programs(0) - 1,
                             h == pl.num_programs(1) - 1))
    def _done():
        out_ref[0] = jnp.sum(acc_ref[...]) * (1.0 / n_total)


def kernel(output, target):
    B, C, H, W = output.shape
    target = target.astype(jnp.int32)

    HS = 2
    mean = pl.pallas_call(
        functools.partial(_mse_onehot_kernel, float(B * C * H * W)),
        grid=(B, HS),
        in_specs=[
            pl.BlockSpec((1, C, H // HS, W), lambda b, h: (b, 0, h, 0)),
            pl.BlockSpec((1, H // HS, W), lambda b, h: (b, h, 0)),
        ],
        out_specs=pl.BlockSpec(memory_space=pltpu.SMEM),
        out_shape=jax.ShapeDtypeStruct((1,), jnp.float32),
        scratch_shapes=[pltpu.VMEM((1, W), jnp.float32)],
        compiler_params=pltpu.CompilerParams(vmem_limit_bytes=100 * 1024 * 1024),
    )(output, target)

    return mean[0]
